# Initial kernel scaffold; baseline (speedup 1.0000x reference)
import jax
import jax.numpy as jnp
from jax.experimental import pallas as pl


def kernel(x, edge_index, edge_attr, edge_indices, edge_attrs, edge_indices_f2c, position, node_attrs, clusters, params):
    raise NotImplementedError("write your pallas kernel here")


# speedup vs baseline: 8.0676x; 8.0676x over previous
import functools
import jax
import jax.numpy as jnp
from jax import lax
from jax.experimental import pallas as pl
from jax.experimental.pallas import tpu as pltpu
from jax.experimental.pallas import tpu_sc as plsc

H = 128
NCORE = 2
NSUB = 16
NWORK = NCORE * NSUB
LW = 128


def _mesh():
    return plsc.VectorSubcoreMesh(core_axis_name="c", subcore_axis_name="s")


def _elu(t):
    return jnp.where(t > 0, t, jnp.exp(jnp.minimum(t, 0.0)) - 1.0)


def _dinv_of(cnt, c):
    deg = c * cnt + 1.0
    safe = jnp.where(deg > 0, deg, 1.0)
    return jnp.where(deg > 0, lax.rsqrt(safe), 0.0)


@functools.lru_cache(maxsize=None)
def _make_counts(e_pad, n_acc):
    k = e_pad // (NWORK * LW)
    z = n_acc // NSUB

    @functools.partial(
        pl.kernel,
        out_type=jax.ShapeDtypeStruct((NCORE, n_acc, H), jnp.float32),
        mesh=_mesh(),
        scratch_types=[
            pltpu.VMEM((k, LW), jnp.int32),
            pltpu.VMEM((LW, H), jnp.float32),
            pltpu.VMEM_SHARED((n_acc, H), jnp.float32),
        ],
    )
    def body(dst_hbm, ones_hbm, zeros_hbm, out_hbm, dst_v, ones_v, acc_sh):
        cid = lax.axis_index("c")
        sid = lax.axis_index("s")
        wid = sid * NCORE + cid
        pltpu.sync_copy(zeros_hbm, acc_sh.at[pl.ds(sid * z, z)])
        pltpu.sync_copy(ones_hbm, ones_v)
        pltpu.sync_copy(dst_hbm.at[wid], dst_v)
        plsc.subcore_barrier()

        def step(j, carry):
            pltpu.sync_copy(ones_v, acc_sh.at[dst_v.at[j]], add=True)
            return carry

        lax.fori_loop(0, k, step, 0)
        plsc.subcore_barrier()
        pltpu.sync_copy(acc_sh.at[pl.ds(sid * z, z)],
                        out_hbm.at[cid, pl.ds(sid * z, z)])

    return body


@functools.lru_cache(maxsize=None)
def _make_gather_scatter(e_pad, n_acc):
    k = e_pad // (NWORK * LW)
    z = n_acc // NSUB

    @functools.partial(
        pl.kernel,
        out_type=jax.ShapeDtypeStruct((NCORE, n_acc, H), jnp.float32),
        mesh=_mesh(),
        scratch_types=[
            pltpu.VMEM((k, LW), jnp.int32),
            pltpu.VMEM((k, LW), jnp.int32),
            pltpu.VMEM((LW, H), jnp.float32),
            pltpu.VMEM_SHARED((n_acc, H), jnp.float32),
            pltpu.SemaphoreType.DMA,
        ],
    )
    def body(u_hbm, src_hbm, dst_hbm, zeros_hbm, out_hbm,
             src_v, dst_v, rows_v, acc_sh, sem):
        cid = lax.axis_index("c")
        sid = lax.axis_index("s")
        wid = sid * NCORE + cid
        pltpu.sync_copy(zeros_hbm, acc_sh.at[pl.ds(sid * z, z)])
        pltpu.sync_copy(src_hbm.at[wid], src_v)
        pltpu.sync_copy(dst_hbm.at[wid], dst_v)
        plsc.subcore_barrier()

        def step(j, carry):
            pltpu.async_copy(u_hbm.at[src_v.at[j]], rows_v, sem).wait()
            pltpu.sync_copy(rows_v, acc_sh.at[dst_v.at[j]], add=True)
            return carry

        lax.fori_loop(0, k, step, 0)
        plsc.subcore_barrier()
        pltpu.sync_copy(acc_sh.at[pl.ds(sid * z, z)],
                        out_hbm.at[cid, pl.ds(sid * z, z)])

    return body


@functools.lru_cache(maxsize=None)
def _make_eac_gather(e_pad):
    ew = e_pad // NWORK
    kc = ew // LW

    @functools.partial(
        pl.kernel,
        out_type=(jax.ShapeDtypeStruct((e_pad, H), jnp.float32),
                  jax.ShapeDtypeStruct((e_pad, H), jnp.float32),
                  jax.ShapeDtypeStruct((e_pad, H), jnp.float32)),
        mesh=_mesh(),
        scratch_types=[
            pltpu.VMEM((kc, LW), jnp.int32),
            pltpu.VMEM((kc, LW), jnp.int32),
            pltpu.VMEM((kc, LW), jnp.int32),
            pltpu.VMEM((LW, H), jnp.float32),
            pltpu.VMEM((LW, H), jnp.float32),
            pltpu.VMEM((LW, H), jnp.float32),
            pltpu.SemaphoreType.DMA,
        ],
    )
    def body(pc_hbm, pf_hbm, src_hbm, dst_hbm, clus_hbm, xc_hbm,
             g1_hbm, g2_hbm, xg_hbm,
             src_v, dst_v, clus_v, pa_v, pb_v, rows_v, sem):
        cid = lax.axis_index("c")
        sid = lax.axis_index("s")
        wid = sid * NCORE + cid
        pltpu.sync_copy(src_hbm.at[wid], src_v)
        pltpu.sync_copy(dst_hbm.at[wid], dst_v)
        pltpu.sync_copy(clus_hbm.at[wid], clus_v)

        def step(j, carry):
            base = wid * ew + j * LW
            pltpu.async_copy(pc_hbm.at[src_v.at[j]], pa_v, sem).wait()
            pltpu.sync_copy(pa_v, g1_hbm.at[pl.ds(base, LW)])
            pltpu.async_copy(pf_hbm.at[dst_v.at[j]], pb_v, sem).wait()
            pltpu.sync_copy(pb_v, g2_hbm.at[pl.ds(base, LW)])
            pltpu.async_copy(xc_hbm.at[clus_v.at[j]], rows_v, sem).wait()
            pltpu.sync_copy(rows_v, xg_hbm.at[pl.ds(base, LW)])
            return carry

        lax.fori_loop(0, kc, step, 0)

    return body


def _tc_call(body, grid, in_specs, out_specs, out_shape):
    return pl.pallas_call(body, grid=grid, in_specs=in_specs,
                          out_specs=out_specs, out_shape=out_shape)


def _edge_const(layers):
    (w1, b1), (w2, b2) = layers
    w1r = w1.reshape(1, H)
    b1r = b1.reshape(1, H)
    w2t = w2.reshape(H, 1).T.reshape(1, H)
    b2r = b2.reshape(1, 1)

    def body(w1_ref, b1_ref, w2_ref, b2_ref, o_ref):
        h = _elu(w1_ref[...] + b1_ref[...])
        s = jnp.sum(h * w2_ref[...], axis=1, keepdims=True) + b2_ref[...]
        o_ref[...] = _elu(s)

    spec = pl.BlockSpec((1, H), lambda: (0, 0))
    spec1 = pl.BlockSpec((1, 1), lambda: (0, 0))
    return pl.pallas_call(
        body,
        in_specs=[spec, spec, spec, spec1],
        out_specs=spec1,
        out_shape=jax.ShapeDtypeStruct((1, 1), jnp.float32),
    )(w1r, b1r, w2t, b2r)


def _gcn_pre(x, w, cnt2, c, blk=512):
    n = x.shape[0]

    def body(x_ref, w_ref, cnt_ref, c_ref, z_ref, u_ref):
        cnt = cnt_ref[0, :, :1] + cnt_ref[1, :, :1]
        dinv = _dinv_of(cnt, c_ref[0, 0])
        z = jnp.dot(x_ref[...], w_ref[...], preferred_element_type=jnp.float32)
        z_ref[...] = z
        u_ref[...] = dinv * z

    o = jax.ShapeDtypeStruct((n, H), jnp.float32)
    return pl.pallas_call(
        body,
        grid=(n // blk,),
        in_specs=[pl.BlockSpec((blk, H), lambda i: (i, 0)),
                  pl.BlockSpec((H, H), lambda i: (0, 0)),
                  pl.BlockSpec((2, blk, H), lambda i: (0, i, 0)),
                  pl.BlockSpec((1, 1), lambda i: (0, 0))],
        out_specs=[pl.BlockSpec((blk, H), lambda i: (i, 0))] * 2,
        out_shape=[o, o],
    )(x, w, cnt2, c)


def _gcn_pre_mean(s2, cntm2, w, cnt2, c, blk=512):
    n = s2.shape[1]

    def body(s_ref, cm_ref, w_ref, cnt_ref, c_ref, x_ref, z_ref, u_ref):
        cm = cm_ref[0, :, :1] + cm_ref[1, :, :1]
        x = (s_ref[0] + s_ref[1]) / jnp.maximum(cm, 1.0)
        cnt = cnt_ref[0, :, :1] + cnt_ref[1, :, :1]
        dinv = _dinv_of(cnt, c_ref[0, 0])
        z = jnp.dot(x, w_ref[...], preferred_element_type=jnp.float32)
        x_ref[...] = x
        z_ref[...] = z
        u_ref[...] = dinv * z

    o = jax.ShapeDtypeStruct((n, H), jnp.float32)
    return pl.pallas_call(
        body,
        grid=(n // blk,),
        in_specs=[pl.BlockSpec((2, blk, H), lambda i: (0, i, 0)),
                  pl.BlockSpec((2, blk, H), lambda i: (0, i, 0)),
                  pl.BlockSpec((H, H), lambda i: (0, 0)),
                  pl.BlockSpec((2, blk, H), lambda i: (0, i, 0)),
                  pl.BlockSpec((1, 1), lambda i: (0, 0))],
        out_specs=[pl.BlockSpec((blk, H), lambda i: (i, 0))] * 3,
        out_shape=[o, o, o],
    )(s2, cntm2, w, cnt2, c)


def _gcn_post(agg2, z, cnt2, c, b, blk=512):
    n = z.shape[0]
    br = b.reshape(1, H)

    def body(a_ref, z_ref, cnt_ref, c_ref, b_ref, o_ref):
        cnt = cnt_ref[0, :, :1] + cnt_ref[1, :, :1]
        cc = c_ref[0, 0]
        dinv = _dinv_of(cnt, cc)
        agg = a_ref[0] + a_ref[1]
        y = dinv * (cc * agg) + (dinv * dinv) * z_ref[...] + b_ref[...]
        o_ref[...] = _elu(y)

    return pl.pallas_call(
        body,
        grid=(n // blk,),
        in_specs=[pl.BlockSpec((2, blk, H), lambda i: (0, i, 0)),
                  pl.BlockSpec((blk, H), lambda i: (i, 0)),
                  pl.BlockSpec((2, blk, H), lambda i: (0, i, 0)),
                  pl.BlockSpec((1, 1), lambda i: (0, 0)),
                  pl.BlockSpec((1, H), lambda i: (0, 0))],
        out_specs=pl.BlockSpec((blk, H), lambda i: (i, 0)),
        out_shape=jax.ShapeDtypeStruct((n, H), jnp.float32),
    )(agg2, z, cnt2, c, br)


def _pos_proj(pc2, pf2, wc1, blk=512):
    n = pc2.shape[0]

    def body(a_ref, b_ref, w_ref, pa_ref, pb_ref):
        f32 = jnp.float32
        pa_ref[...] = jnp.dot(a_ref[...] * 4.0, w_ref[...],
                              preferred_element_type=f32)
        pb_ref[...] = jnp.dot(b_ref[...] * 4.0, w_ref[...],
                              preferred_element_type=f32)

    o = jax.ShapeDtypeStruct((n, H), jnp.float32)
    return pl.pallas_call(
        body,
        grid=(n // blk,),
        in_specs=[pl.BlockSpec((blk, 2), lambda i: (i, 0)),
                  pl.BlockSpec((blk, 2), lambda i: (i, 0)),
                  pl.BlockSpec((2, H), lambda i: (0, 0))],
        out_specs=[pl.BlockSpec((blk, H), lambda i: (i, 0))] * 2,
        out_shape=[o, o],
    )(pc2, pf2, wc1)


def _c2f_up(g1, g2, xg, pc2f, pup, pnorm, blk=512):
    n = g1.shape[0]
    (wc1, bc1), (wc2, bc2) = pc2f
    (wu1, bu1), (wu2, bu2) = pup
    g, beta = pnorm

    def body(g1_ref, g2_ref, x_ref, bc1_r, wc2_r, bc2_r,
             wu1_r, bu1_r, wu2_r, bu2_r, g_r, be_r, o_ref):
        f32 = jnp.float32
        eh = _elu(g1_ref[...] - g2_ref[...] + bc1_r[...])
        eac = _elu(jnp.dot(eh, wc2_r[...], preferred_element_type=f32)
                   + bc2_r[...])
        t_in = jnp.concatenate([eac, x_ref[...]], axis=1)
        th = _elu(jnp.dot(t_in, wu1_r[...], preferred_element_type=f32)
                  + bu1_r[...])
        t2 = _elu(jnp.dot(th, wu2_r[...], preferred_element_type=f32)
                  + bu2_r[...])
        t = eac + t2
        mu = jnp.mean(t, axis=1, keepdims=True)
        var = jnp.mean((t - mu) ** 2, axis=1, keepdims=True)
        t = (t - mu) / jnp.sqrt(var + 1e-5) * g_r[...] + be_r[...]
        o_ref[...] = t

    row = lambda a: a.reshape(1, -1)
    full = lambda s: pl.BlockSpec(s, lambda i: (0, 0))
    return pl.pallas_call(
        body,
        grid=(n // blk,),
        in_specs=[pl.BlockSpec((blk, H), lambda i: (i, 0)),
                  pl.BlockSpec((blk, H), lambda i: (i, 0)),
                  pl.BlockSpec((blk, H), lambda i: (i, 0)),
                  full((1, H)), full((H, H)), full((1, H)),
                  full((2 * H, H)), full((1, H)), full((H, H)), full((1, H)),
                  full((1, H)), full((1, H))],
        out_specs=pl.BlockSpec((blk, H), lambda i: (i, 0)),
        out_shape=jax.ShapeDtypeStruct((n, H), jnp.float32),
    )(g1, g2, xg, row(bc1), wc2, row(bc2),
      wu1, row(bu1), wu2, row(bu2), row(g), row(beta))


def _node_dec(x, layers, blk=512):
    (w1, b1), (w2, b2) = layers
    n = x.shape[0]
    out_f = w2.shape[1]
    w2p = jnp.zeros((H, H), jnp.float32).at[:, :out_f].set(w2)
    b2p = jnp.zeros((1, H), jnp.float32).at[0, :out_f].set(b2)

    def body(x_ref, w1_r, b1_r, w2_r, b2_r, o_ref):
        f32 = jnp.float32
        h = _elu(jnp.dot(x_ref[...], w1_r[...], preferred_element_type=f32)
                 + b1_r[...])
        o_ref[...] = _elu(jnp.dot(h, w2_r[...], preferred_element_type=f32)
                          + b2_r[...])

    full = lambda s: pl.BlockSpec(s, lambda i: (0, 0))
    return pl.pallas_call(
        body,
        grid=(n // blk,),
        in_specs=[pl.BlockSpec((blk, H), lambda i: (i, 0)),
                  full((H, H)), full((1, H)), full((H, H)), full((1, H))],
        out_specs=pl.BlockSpec((blk, H), lambda i: (i, 0)),
        out_shape=jax.ShapeDtypeStruct((n, H), jnp.float32),
    )(x, w1, b1.reshape(1, H), w2p, b2p)


def _fill_const(c, rows, cols):
    def body(c_ref, o_ref):
        o_ref[...] = jnp.full((rows, cols), 1.0, jnp.float32) * c_ref[0, 0]

    return pl.pallas_call(
        body,
        in_specs=[pl.BlockSpec((1, 1), lambda: (0, 0))],
        out_specs=pl.BlockSpec((rows, cols), lambda: (0, 0)),
        out_shape=jax.ShapeDtypeStruct((rows, cols), jnp.float32),
    )(c)


def _pad1d(a, e_pad, fill):
    a = a.astype(jnp.int32)
    return jnp.concatenate(
        [a, jnp.full((e_pad - a.shape[0],), fill, jnp.int32)])


def _pad3d(a, e_pad, fill):
    return _pad1d(a, e_pad, fill).reshape(NWORK, e_pad // (NWORK * LW), LW)


def _ceil_to(v, m):
    return ((v + m - 1) // m) * m


def kernel(x, edge_index, edge_attr, edge_indices, edge_attrs,
           edge_indices_f2c, position, node_attrs, clusters, params):
    nc = x.shape[0]
    nf = position.shape[1]
    ec = edge_index.shape[1]
    ef = edge_indices.shape[2]

    na0 = _ceil_to(nc + 1, 512)
    na1 = _ceil_to(nf + 1, 512)
    ecp = _ceil_to(ec, NWORK * LW)
    efp = _ceil_to(ef, NWORK * LW)
    nmp = _ceil_to(nf, NWORK * LW)

    x0 = jnp.zeros((na0, H), jnp.float32).at[:nc].set(x)
    src_c = _pad3d(edge_index[0], ecp, 0)
    dst_c = _pad3d(edge_index[1], ecp, nc)
    ei = edge_indices[0]
    src_f = _pad3d(ei[0], efp, 0)
    dst_f = _pad3d(ei[1], efp, nf)
    f2c = edge_indices_f2c[0]
    c2f_src3 = _pad3d(f2c[1], nmp, 0)
    c2f_dst3 = _pad3d(f2c[0], nmp, 0)
    mean_src = _pad3d(jnp.arange(nf, dtype=jnp.int32), nmp, 0)
    mean_dst = _pad3d(f2c[0], nmp, nf)
    clus2d = _pad3d(clusters[0], nmp, 0)
    pf2 = position[0][:na0].astype(jnp.float32)
    pc2 = position[1][:na0].astype(jnp.float32)

    ones128 = jnp.ones((LW, H), jnp.float32)
    zc128 = jnp.zeros((na0 // NSUB, H), jnp.float32)
    zf128 = jnp.zeros((na1 // NSUB, H), jnp.float32)

    c0 = _edge_const(params['edge_dec'][0])
    c1 = _edge_const(params['edge_dec'][1])

    cnt_c = _make_counts(ecp, na0)(dst_c, ones128, zc128)
    cnt_f = _make_counts(efp, na1)(dst_f, ones128, zf128)
    cnt_m = _make_counts(nmp, na1)(mean_dst, ones128, zf128)

    gs_c = _make_gather_scatter(ecp, na0)
    xc = x0
    for w, b in params['mp'][0]:
        z, u = _gcn_pre(xc, w, cnt_c, c0)
        agg = gs_c(u, src_c, dst_c, zc128)
        xc = _gcn_post(agg, z, cnt_c, c0, b)

    pa, pb = _pos_proj(pc2, pf2, params['c2f'][0][0])
    g1, g2, xg = _make_eac_gather(nmp)(
        pa, pb, c2f_src3, c2f_dst3, clus2d, xc)
    t = _c2f_up(g1, g2, xg, params['c2f'], params['up'], params['up_norm'])
    s2 = _make_gather_scatter(nmp, na1)(t, mean_src, mean_dst, zf128)

    gs_f = _make_gather_scatter(efp, na1)
    (w1, b1), (w2, b2) = params['mp'][1]
    xf, z, u = _gcn_pre_mean(s2, cnt_m, w1, cnt_f, c1)
    agg = gs_f(u, src_f, dst_f, zf128)
    xf = _gcn_post(agg, z, cnt_f, c1, b1)
    z, u = _gcn_pre(xf, w2, cnt_f, c1)
    agg = gs_f(u, src_f, dst_f, zf128)
    xf = _gcn_post(agg, z, cnt_f, c1, b2)

    y = _node_dec(xf, params['node_dec'])
    out_f = params['node_dec'][1][0].shape[1]
    y = y[:nf, :out_f]
    ea2 = _fill_const(c1, ef // LW, LW).reshape(ef, 1)
    return (y, ei, ea2)
